# Initial kernel scaffold; baseline (speedup 1.0000x reference)
import numpy as np
import jax
import jax.numpy as jnp
from jax import lax
from jax.experimental import pallas as pl
from jax.experimental.pallas import tpu as pltpu
from jax.experimental.pallas import tpu_sc as plsc

H = 1024
W = 1024
L = 16
T = 2 ** 19
F = 2
BASE_RES = 16
FINEST_RES = 2048
_b = np.exp((np.log(FINEST_RES) - np.log(BASE_RES)) / (L - 1))
RESOLUTIONS = [int(np.floor(BASE_RES * _b ** l)) for l in range(L)]
RESIDUAL_SCALE = 0.25
P1S = int(np.uint32(2654435761).astype(np.int64)) - (1 << 32)
P2S = 805459861
HMASK = T - 1

NC = 2
NS = 16
NWORK = NC * NS
PTS = H * W
PER_W = PTS // NWORK
CH = 2048
NCHUNK = PER_W // CH
TOTCH = PTS // CH
NG16 = CH // 16
NG8 = CH // 8


def _sc_body(tbl, enc,
             idx00, idx10, idx01, idx11,
             rows00, rows10, rows01, rows11,
             encv, sem):
    wid = lax.axis_index("s") * NC + lax.axis_index("c")
    iota = lax.iota(jnp.int32, 16)
    half8 = iota >> 1
    par = iota & 1

    def chunk_body(ci, carry):
        pbase = wid * PER_W + ci * CH
        for l in range(L):
            res = RESOLUTIONS[l]
            scale = np.float32(res / 1024.0)
            base_l = l * T

            def idx_body(g, c):
                pid = pbase + g * 16 + iota
                x = pid & (W - 1)
                y = pid >> 10
                x0 = (x.astype(jnp.float32) * scale).astype(jnp.int32)
                y0 = (y.astype(jnp.float32) * scale).astype(jnp.int32)
                hx0 = x0 * P1S
                hx1 = (x0 + 1) * P1S
                hy0 = y0 * P2S
                hy1 = (y0 + 1) * P2S
                sl = pl.ds(g * 16, 16)
                idx00[sl] = ((hx0 ^ hy0) & HMASK) + base_l
                idx10[sl] = ((hx1 ^ hy0) & HMASK) + base_l
                idx01[sl] = ((hx0 ^ hy1) & HMASK) + base_l
                idx11[sl] = ((hx1 ^ hy1) & HMASK) + base_l
                return c

            lax.fori_loop(0, NG16, idx_body, 0)
            c00 = pltpu.async_copy(tbl.at[idx00], rows00, sem)
            c10 = pltpu.async_copy(tbl.at[idx10], rows10, sem)
            c01 = pltpu.async_copy(tbl.at[idx01], rows01, sem)
            c11 = pltpu.async_copy(tbl.at[idx11], rows11, sem)
            c00.wait()
            c10.wait()
            c01.wait()
            c11.wait()

            def blend_body(g, c):
                pt = g * 8 + half8
                pid = pbase + pt
                x = pid & (W - 1)
                y = pid >> 10
                posx = x.astype(jnp.float32) * scale
                posy = y.astype(jnp.float32) * scale
                x0f = posx.astype(jnp.int32).astype(jnp.float32)
                y0f = posy.astype(jnp.int32).astype(jnp.float32)
                wx = posx - x0f
                wy = posy - y0f
                ex = 1.0 - wx
                ey = 1.0 - wy
                f00 = plsc.load_gather(rows00, [pt, par])
                f10 = plsc.load_gather(rows10, [pt, par])
                f01 = plsc.load_gather(rows01, [pt, par])
                f11 = plsc.load_gather(rows11, [pt, par])
                f = (f00 * ex + f10 * wx) * ey + (f01 * ex + f11 * wx) * wy
                plsc.store_scatter(encv, [2 * l + par, pt], f)
                return c

            lax.fori_loop(0, NG8, blend_body, 0)
        pltpu.sync_copy(encv, enc.at[wid * NCHUNK + ci])
        return carry

    lax.fori_loop(0, NCHUNK, chunk_body, 0)


def _sc_encode(tbl):
    mesh = plsc.VectorSubcoreMesh(core_axis_name="c", subcore_axis_name="s")
    fn = pl.kernel(
        _sc_body,
        out_type=jax.ShapeDtypeStruct((TOTCH, 32, CH), jnp.float32),
        mesh=mesh,
        scratch_types=[
            pltpu.VMEM((CH,), jnp.int32),
            pltpu.VMEM((CH,), jnp.int32),
            pltpu.VMEM((CH,), jnp.int32),
            pltpu.VMEM((CH,), jnp.int32),
            pltpu.VMEM((CH, F), jnp.float32),
            pltpu.VMEM((CH, F), jnp.float32),
            pltpu.VMEM((CH, F), jnp.float32),
            pltpu.VMEM((CH, F), jnp.float32),
            pltpu.VMEM((32, CH), jnp.float32),
            pltpu.SemaphoreType.DMA,
        ],
    )
    return fn(tbl)


ROWS_PER = CH // W


def _tc_body(enc_ref, tex_ref, w1_ref, b1_ref, w2_ref, b2_ref,
             w3_ref, b3_ref, out_ref):
    e = enc_ref[0].astype(jnp.bfloat16)
    h1 = jnp.dot(w1_ref[...], e, preferred_element_type=jnp.float32)
    h1 = jnp.maximum(h1 + b1_ref[...], 0.0).astype(jnp.bfloat16)
    h2 = jnp.dot(w2_ref[...], h1, preferred_element_type=jnp.float32)
    h2 = jnp.maximum(h2 + b2_ref[...], 0.0).astype(jnp.bfloat16)
    r = jnp.dot(w3_ref[...], h2, preferred_element_type=jnp.float32)
    r = r + b3_ref[...]
    resid = RESIDUAL_SCALE * jnp.tanh(r)
    base = 1.0 / (1.0 + jnp.exp(-tex_ref[0]))
    for rr in range(ROWS_PER):
        o = base[:, rr, :] + resid[0:3, rr * W:(rr + 1) * W]
        out_ref[0, :, rr, :] = jnp.clip(o, 0.0, 1.0)


def _tc_mlp(enc, texture_map, w1t, b1c, w2t, b2c, w3t, b3c):
    return pl.pallas_call(
        _tc_body,
        grid=(TOTCH,),
        in_specs=[
            pl.BlockSpec((1, 32, CH), lambda i: (i, 0, 0)),
            pl.BlockSpec((1, 3, ROWS_PER, W), lambda i: (0, 0, i, 0)),
            pl.BlockSpec((64, 32), lambda i: (0, 0)),
            pl.BlockSpec((64, 1), lambda i: (0, 0)),
            pl.BlockSpec((64, 64), lambda i: (0, 0)),
            pl.BlockSpec((64, 1), lambda i: (0, 0)),
            pl.BlockSpec((8, 64), lambda i: (0, 0)),
            pl.BlockSpec((8, 1), lambda i: (0, 0)),
        ],
        out_specs=pl.BlockSpec((1, 3, ROWS_PER, W), lambda i: (0, 0, i, 0)),
        out_shape=jax.ShapeDtypeStruct((1, 3, H, W), jnp.float32),
    )(enc, texture_map, w1t, b1c, w2t, b2c, w3t, b3c)


def kernel(texture_map, hash_tables, W1, b1, W2, b2, W3, b3):
    tbl = hash_tables.reshape(L * T, F)
    enc = _sc_encode(tbl)
    w1t = W1.T.astype(jnp.bfloat16)
    w2t = W2.T.astype(jnp.bfloat16)
    w3t = jnp.concatenate([W3.T, jnp.zeros((5, 64), W3.dtype)], axis=0)
    w3t = w3t.astype(jnp.bfloat16)
    b1c = b1.reshape(64, 1)
    b2c = b2.reshape(64, 1)
    b3c = jnp.concatenate([b3, jnp.zeros((5,), b3.dtype)]).reshape(8, 1)
    return _tc_mlp(enc, texture_map, w1t, b1c, w2t, b2c, w3t, b3c)


# speedup vs baseline: 30.3311x; 30.3311x over previous
import numpy as np
import jax
import jax.numpy as jnp
from jax import lax
from jax.experimental import pallas as pl
from jax.experimental.pallas import tpu as pltpu
from jax.experimental.pallas import tpu_sc as plsc

H = 1024
W = 1024
L = 16
T = 2 ** 19
F = 2
BASE_RES = 16
FINEST_RES = 2048
_b = np.exp((np.log(FINEST_RES) - np.log(BASE_RES)) / (L - 1))
RESOLUTIONS = [int(np.floor(BASE_RES * _b ** l)) for l in range(L)]
RESIDUAL_SCALE = 0.25
P1S = int(np.uint32(2654435761).astype(np.int64)) - (1 << 32)
P2S = 805459861
HMASK = T - 1

NC = 2
NS = 16
NWORK = NC * NS
PTS = H * W
PER_W = PTS // NWORK
CH = 1024
NCHUNK = PER_W // CH
TOTCH = PTS // CH
NG16 = CH // 16
NG8 = CH // 8


def _sc_body(tbl, enc,
             idx00, idx10, idx01, idx11,
             rows00, rows10, rows01, rows11,
             encv, sem):
    wid = lax.axis_index("s") * NC + lax.axis_index("c")
    iota = lax.iota(jnp.int32, 16)
    half8 = iota >> 1
    par = iota & 1

    def chunk_body(ci, carry):
        pbase = wid * PER_W + ci * CH
        for l in range(L):
            res = RESOLUTIONS[l]
            scale = np.float32(res / 1024.0)
            base_l = l * T

            def idx_body(g, c):
                pid = pbase + g * 16 + iota
                x = pid & (W - 1)
                y = pid >> 10
                x0 = (x.astype(jnp.float32) * scale).astype(jnp.int32)
                y0 = (y.astype(jnp.float32) * scale).astype(jnp.int32)
                hx0 = x0 * P1S
                hx1 = (x0 + 1) * P1S
                hy0 = y0 * P2S
                hy1 = (y0 + 1) * P2S
                sl = pl.ds(g * 16, 16)
                idx00[sl] = ((hx0 ^ hy0) & HMASK) + base_l
                idx10[sl] = ((hx1 ^ hy0) & HMASK) + base_l
                idx01[sl] = ((hx0 ^ hy1) & HMASK) + base_l
                idx11[sl] = ((hx1 ^ hy1) & HMASK) + base_l
                return c

            lax.fori_loop(0, NG16, idx_body, 0)
            c00 = pltpu.async_copy(tbl.at[idx00], rows00, sem)
            c10 = pltpu.async_copy(tbl.at[idx10], rows10, sem)
            c01 = pltpu.async_copy(tbl.at[idx01], rows01, sem)
            c11 = pltpu.async_copy(tbl.at[idx11], rows11, sem)
            c00.wait()
            c10.wait()
            c01.wait()
            c11.wait()

            def blend_body(g, c):
                pt = g * 8 + half8
                pid = pbase + pt
                x = pid & (W - 1)
                y = pid >> 10
                posx = x.astype(jnp.float32) * scale
                posy = y.astype(jnp.float32) * scale
                x0f = posx.astype(jnp.int32).astype(jnp.float32)
                y0f = posy.astype(jnp.int32).astype(jnp.float32)
                wx = posx - x0f
                wy = posy - y0f
                ex = 1.0 - wx
                ey = 1.0 - wy
                f00 = plsc.load_gather(rows00, [pt, par])
                f10 = plsc.load_gather(rows10, [pt, par])
                f01 = plsc.load_gather(rows01, [pt, par])
                f11 = plsc.load_gather(rows11, [pt, par])
                f = (f00 * ex + f10 * wx) * ey + (f01 * ex + f11 * wx) * wy
                plsc.store_scatter(encv, [2 * l + par, pt], f)
                return c

            lax.fori_loop(0, NG8, blend_body, 0)
        pltpu.sync_copy(encv, enc.at[wid * NCHUNK + ci])
        return carry

    lax.fori_loop(0, NCHUNK, chunk_body, 0)


def _sc_encode(tbl):
    mesh = plsc.VectorSubcoreMesh(core_axis_name="c", subcore_axis_name="s")
    fn = pl.kernel(
        _sc_body,
        out_type=jax.ShapeDtypeStruct((TOTCH, 32, CH), jnp.float32),
        mesh=mesh,
        compiler_params=pltpu.CompilerParams(needs_layout_passes=False, use_tc_tiling_on_sc=False),
        scratch_types=[
            pltpu.VMEM((CH,), jnp.int32),
            pltpu.VMEM((CH,), jnp.int32),
            pltpu.VMEM((CH,), jnp.int32),
            pltpu.VMEM((CH,), jnp.int32),
            pltpu.VMEM((CH, F), jnp.float32),
            pltpu.VMEM((CH, F), jnp.float32),
            pltpu.VMEM((CH, F), jnp.float32),
            pltpu.VMEM((CH, F), jnp.float32),
            pltpu.VMEM((32, CH), jnp.float32),
            pltpu.SemaphoreType.DMA,
        ],
    )
    return fn(tbl)


ROWS_PER = CH // W


CPB = 8
ROWS_TC = CPB * ROWS_PER


def _tc_body(enc_ref, tex_ref, w1_ref, b1_ref, w2_ref, b2_ref,
             w3_ref, b3_ref, out_ref):
    base = 1.0 / (1.0 + jnp.exp(-tex_ref[0]))
    for s in range(CPB):
        e = enc_ref[s].astype(jnp.bfloat16)
        h1 = jnp.dot(w1_ref[...], e, preferred_element_type=jnp.float32)
        h1 = jnp.maximum(h1 + b1_ref[...], 0.0).astype(jnp.bfloat16)
        h2 = jnp.dot(w2_ref[...], h1, preferred_element_type=jnp.float32)
        h2 = jnp.maximum(h2 + b2_ref[...], 0.0).astype(jnp.bfloat16)
        r = jnp.dot(w3_ref[...], h2, preferred_element_type=jnp.float32)
        r = r + b3_ref[...]
        resid = RESIDUAL_SCALE * jnp.tanh(r)
        for rr in range(ROWS_PER):
            row = s * ROWS_PER + rr
            o = base[:, row, :] + resid[0:3, rr * W:(rr + 1) * W]
            out_ref[0, :, row, :] = jnp.clip(o, 0.0, 1.0)


def _tc_mlp(enc, texture_map, w1t, b1c, w2t, b2c, w3t, b3c):
    return pl.pallas_call(
        _tc_body,
        grid=(TOTCH // CPB,),
        in_specs=[
            pl.BlockSpec((CPB, 32, CH), lambda i: (i, 0, 0)),
            pl.BlockSpec((1, 3, ROWS_TC, W), lambda i: (0, 0, i, 0)),
            pl.BlockSpec((64, 32), lambda i: (0, 0)),
            pl.BlockSpec((64, 1), lambda i: (0, 0)),
            pl.BlockSpec((64, 64), lambda i: (0, 0)),
            pl.BlockSpec((64, 1), lambda i: (0, 0)),
            pl.BlockSpec((8, 64), lambda i: (0, 0)),
            pl.BlockSpec((8, 1), lambda i: (0, 0)),
        ],
        out_specs=pl.BlockSpec((1, 3, ROWS_TC, W), lambda i: (0, 0, i, 0)),
        out_shape=jax.ShapeDtypeStruct((1, 3, H, W), jnp.float32),
    )(enc, texture_map, w1t, b1c, w2t, b2c, w3t, b3c)


def kernel(texture_map, hash_tables, W1, b1, W2, b2, W3, b3):
    tbl = hash_tables.reshape(L * T, F)
    enc = _sc_encode(tbl)
    w1t = W1.T.astype(jnp.bfloat16)
    w2t = W2.T.astype(jnp.bfloat16)
    w3t = jnp.concatenate([W3.T, jnp.zeros((5, 64), W3.dtype)], axis=0)
    w3t = w3t.astype(jnp.bfloat16)
    b1c = b1.reshape(64, 1)
    b2c = b2.reshape(64, 1)
    b3c = jnp.concatenate([b3, jnp.zeros((5,), b3.dtype)]).reshape(8, 1)
    return _tc_mlp(enc, texture_map, w1t, b1c, w2t, b2c, w3t, b3c)


# speedup vs baseline: 31.2042x; 1.0288x over previous
import numpy as np
import jax
import jax.numpy as jnp
from jax import lax
from jax.experimental import pallas as pl
from jax.experimental.pallas import tpu as pltpu
from jax.experimental.pallas import tpu_sc as plsc

H = 1024
W = 1024
L = 16
T = 2 ** 19
F = 2
BASE_RES = 16
FINEST_RES = 2048
_b = np.exp((np.log(FINEST_RES) - np.log(BASE_RES)) / (L - 1))
RESOLUTIONS = [int(np.floor(BASE_RES * _b ** l)) for l in range(L)]
RESIDUAL_SCALE = 0.25
P1S = int(np.uint32(2654435761).astype(np.int64)) - (1 << 32)
P2S = 805459861
HMASK = T - 1

NC = 2
NS = 16
NWORK = NC * NS
PTS = H * W
PER_W = PTS // NWORK
CH = 1024
NCHUNK = PER_W // CH
TOTCH = PTS // CH
NG16 = CH // 16


def _sc_body(tbl, enc,
             i00a, i00b, i10a, i10b, i01a, i01b, i11a, i11b,
             r00a, r00b, r10a, r10b, r01a, r01b, r11a, r11b,
             encv, sem):
    wid = lax.axis_index("s") * NC + lax.axis_index("c")
    iota = lax.iota(jnp.int32, 16)

    def chunk_body(ci, carry):
        pbase = wid * PER_W + ci * CH
        for l in range(L):
            res = RESOLUTIONS[l]
            scale = np.float32(res / 1024.0)
            ebase = 2 * l * T

            def idx_body(g, c):
                pid = pbase + g * 16 + iota
                x = pid & (W - 1)
                y = pid >> 10
                x0 = (x.astype(jnp.float32) * scale).astype(jnp.int32)
                y0 = (y.astype(jnp.float32) * scale).astype(jnp.int32)
                hx0 = x0 * P1S
                hx1 = hx0 + P1S
                hy0 = y0 * P2S
                hy1 = hy0 + P2S
                sl = pl.ds(g * 16, 16)
                e00 = (((hx0 ^ hy0) & HMASK) << 1) + ebase
                e10 = (((hx1 ^ hy0) & HMASK) << 1) + ebase
                e01 = (((hx0 ^ hy1) & HMASK) << 1) + ebase
                e11 = (((hx1 ^ hy1) & HMASK) << 1) + ebase
                i00a[sl] = e00
                i00b[sl] = e00 + 1
                i10a[sl] = e10
                i10b[sl] = e10 + 1
                i01a[sl] = e01
                i01b[sl] = e01 + 1
                i11a[sl] = e11
                i11b[sl] = e11 + 1
                return c

            lax.fori_loop(0, NG16, idx_body, 0)
            cps = [
                pltpu.async_copy(tbl.at[i00a], r00a, sem),
                pltpu.async_copy(tbl.at[i00b], r00b, sem),
                pltpu.async_copy(tbl.at[i10a], r10a, sem),
                pltpu.async_copy(tbl.at[i10b], r10b, sem),
                pltpu.async_copy(tbl.at[i01a], r01a, sem),
                pltpu.async_copy(tbl.at[i01b], r01b, sem),
                pltpu.async_copy(tbl.at[i11a], r11a, sem),
                pltpu.async_copy(tbl.at[i11b], r11b, sem),
            ]
            for cp in cps:
                cp.wait()

            def blend_body(g, c):
                pid = pbase + g * 16 + iota
                x = pid & (W - 1)
                y = pid >> 10
                posx = x.astype(jnp.float32) * scale
                posy = y.astype(jnp.float32) * scale
                x0f = posx.astype(jnp.int32).astype(jnp.float32)
                y0f = posy.astype(jnp.int32).astype(jnp.float32)
                wx = posx - x0f
                wy = posy - y0f
                ex = 1.0 - wx
                ey = 1.0 - wy
                sl = pl.ds(g * 16, 16)
                f0 = (r00a[sl] * ex + r10a[sl] * wx) * ey + \
                     (r01a[sl] * ex + r11a[sl] * wx) * wy
                f1 = (r00b[sl] * ex + r10b[sl] * wx) * ey + \
                     (r01b[sl] * ex + r11b[sl] * wx) * wy
                encv[pl.ds(2 * l * CH + g * 16, 16)] = f0
                encv[pl.ds((2 * l + 1) * CH + g * 16, 16)] = f1
                return c

            lax.fori_loop(0, NG16, blend_body, 0)
        pltpu.sync_copy(encv, enc.at[pl.ds((wid * NCHUNK + ci) * 32 * CH, 32 * CH)])
        return carry

    lax.fori_loop(0, NCHUNK, chunk_body, 0)


def _sc_encode(tbl):
    mesh = plsc.VectorSubcoreMesh(core_axis_name="c", subcore_axis_name="s")
    fn = pl.kernel(
        _sc_body,
        out_type=jax.ShapeDtypeStruct((TOTCH * 32 * CH,), jnp.float32),
        mesh=mesh,
        compiler_params=pltpu.CompilerParams(needs_layout_passes=False, use_tc_tiling_on_sc=False),
        scratch_types=[
            pltpu.VMEM((CH,), jnp.int32),
            pltpu.VMEM((CH,), jnp.int32),
            pltpu.VMEM((CH,), jnp.int32),
            pltpu.VMEM((CH,), jnp.int32),
            pltpu.VMEM((CH,), jnp.int32),
            pltpu.VMEM((CH,), jnp.int32),
            pltpu.VMEM((CH,), jnp.int32),
            pltpu.VMEM((CH,), jnp.int32),
            pltpu.VMEM((CH,), jnp.float32),
            pltpu.VMEM((CH,), jnp.float32),
            pltpu.VMEM((CH,), jnp.float32),
            pltpu.VMEM((CH,), jnp.float32),
            pltpu.VMEM((CH,), jnp.float32),
            pltpu.VMEM((CH,), jnp.float32),
            pltpu.VMEM((CH,), jnp.float32),
            pltpu.VMEM((CH,), jnp.float32),
            pltpu.VMEM((32 * CH,), jnp.float32),
            pltpu.SemaphoreType.DMA,
        ],
    )
    return fn(tbl)


ROWS_PER = CH // W


CPB = 8
ROWS_TC = CPB * ROWS_PER


def _tc_body(enc_ref, tex_ref, w1_ref, b1_ref, w2_ref, b2_ref,
             w3_ref, b3_ref, out_ref):
    base = 1.0 / (1.0 + jnp.exp(-tex_ref[0]))
    for s in range(CPB):
        e = enc_ref[pl.ds(s * 32 * CH, 32 * CH)].reshape(32, CH)
        e = e.astype(jnp.bfloat16)
        h1 = jnp.dot(w1_ref[...], e, preferred_element_type=jnp.float32)
        h1 = jnp.maximum(h1 + b1_ref[...], 0.0).astype(jnp.bfloat16)
        h2 = jnp.dot(w2_ref[...], h1, preferred_element_type=jnp.float32)
        h2 = jnp.maximum(h2 + b2_ref[...], 0.0).astype(jnp.bfloat16)
        r = jnp.dot(w3_ref[...], h2, preferred_element_type=jnp.float32)
        r = r + b3_ref[...]
        resid = RESIDUAL_SCALE * jnp.tanh(r)
        for rr in range(ROWS_PER):
            row = s * ROWS_PER + rr
            o = base[:, row, :] + resid[0:3, rr * W:(rr + 1) * W]
            out_ref[0, :, row, :] = jnp.clip(o, 0.0, 1.0)


def _tc_mlp(enc, texture_map, w1t, b1c, w2t, b2c, w3t, b3c):
    return pl.pallas_call(
        _tc_body,
        grid=(TOTCH // CPB,),
        in_specs=[
            pl.BlockSpec((CPB * 32 * CH,), lambda i: (i,)),
            pl.BlockSpec((1, 3, ROWS_TC, W), lambda i: (0, 0, i, 0)),
            pl.BlockSpec((64, 32), lambda i: (0, 0)),
            pl.BlockSpec((64, 1), lambda i: (0, 0)),
            pl.BlockSpec((64, 64), lambda i: (0, 0)),
            pl.BlockSpec((64, 1), lambda i: (0, 0)),
            pl.BlockSpec((8, 64), lambda i: (0, 0)),
            pl.BlockSpec((8, 1), lambda i: (0, 0)),
        ],
        out_specs=pl.BlockSpec((1, 3, ROWS_TC, W), lambda i: (0, 0, i, 0)),
        out_shape=jax.ShapeDtypeStruct((1, 3, H, W), jnp.float32),
    )(enc, texture_map, w1t, b1c, w2t, b2c, w3t, b3c)


def kernel(texture_map, hash_tables, W1, b1, W2, b2, W3, b3):
    tbl = hash_tables.reshape(L * T * F)
    enc = _sc_encode(tbl)
    w1t = W1.T.astype(jnp.bfloat16)
    w2t = W2.T.astype(jnp.bfloat16)
    w3t = jnp.concatenate([W3.T, jnp.zeros((5, 64), W3.dtype)], axis=0)
    w3t = w3t.astype(jnp.bfloat16)
    b1c = b1.reshape(64, 1)
    b2c = b2.reshape(64, 1)
    b3c = jnp.concatenate([b3, jnp.zeros((5,), b3.dtype)]).reshape(8, 1)
    return _tc_mlp(enc, texture_map, w1t, b1c, w2t, b2c, w3t, b3c)


# speedup vs baseline: 84.5390x; 2.7092x over previous
import numpy as np
import jax
import jax.numpy as jnp
from jax import lax
from jax.experimental import pallas as pl
from jax.experimental.pallas import tpu as pltpu
from jax.experimental.pallas import tpu_sc as plsc

H = 1024
W = 1024
L = 16
T = 2 ** 19
F = 2
BASE_RES = 16
FINEST_RES = 2048
_b = np.exp((np.log(FINEST_RES) - np.log(BASE_RES)) / (L - 1))
RESOLUTIONS = [int(np.floor(BASE_RES * _b ** l)) for l in range(L)]
RESIDUAL_SCALE = 0.25
P1S = int(np.uint32(2654435761).astype(np.int64)) - (1 << 32)
P2S = 805459861
HMASK = T - 1

NC = 2
NS = 16
NWORK = NC * NS
PTS = H * W
PER_W = PTS // NWORK
CH = 1024
NCHUNK = PER_W // CH
TOTCH = PTS // CH
NG16 = CH // 16


def _sc_body(tbl, enc,
             i00a, i00b, i10a, i10b, i01a, i01b, i11a, i11b,
             r00a, r00b, r10a, r10b, r01a, r01b, r11a, r11b,
             encv, sem):
    wid = lax.axis_index("s") * NC + lax.axis_index("c")
    iota = lax.iota(jnp.int32, 16)

    def chunk_body(ci, carry):
        pbase = wid * PER_W + ci * CH
        for l in range(L):
            res = RESOLUTIONS[l]
            scale = np.float32(res / 1024.0)
            ebase = 2 * l * T

            def idx_body(g, c):
                pid = pbase + g * 16 + iota
                x = pid & (W - 1)
                y = pid >> 10
                x0 = (x.astype(jnp.float32) * scale).astype(jnp.int32)
                y0 = (y.astype(jnp.float32) * scale).astype(jnp.int32)
                hx0 = x0 * P1S
                hx1 = hx0 + P1S
                hy0 = y0 * P2S
                hy1 = hy0 + P2S
                sl = pl.ds(g * 16, 16)
                h00 = (hx0 ^ hy0) & HMASK
                h10 = (hx1 ^ hy0) & HMASK
                h01 = (hx0 ^ hy1) & HMASK
                h11 = (hx1 ^ hy1) & HMASK
                e00 = ((h00 << 1) - (h00 & 127)) + ebase
                e10 = ((h10 << 1) - (h10 & 127)) + ebase
                e01 = ((h01 << 1) - (h01 & 127)) + ebase
                e11 = ((h11 << 1) - (h11 & 127)) + ebase
                i00a[sl] = e00
                i00b[sl] = e00 + 128
                i10a[sl] = e10
                i10b[sl] = e10 + 128
                i01a[sl] = e01
                i01b[sl] = e01 + 128
                i11a[sl] = e11
                i11b[sl] = e11 + 128
                return c

            lax.fori_loop(0, NG16, idx_body, 0)
            cps = [
                pltpu.async_copy(tbl.at[i00a], r00a, sem),
                pltpu.async_copy(tbl.at[i00b], r00b, sem),
                pltpu.async_copy(tbl.at[i10a], r10a, sem),
                pltpu.async_copy(tbl.at[i10b], r10b, sem),
                pltpu.async_copy(tbl.at[i01a], r01a, sem),
                pltpu.async_copy(tbl.at[i01b], r01b, sem),
                pltpu.async_copy(tbl.at[i11a], r11a, sem),
                pltpu.async_copy(tbl.at[i11b], r11b, sem),
            ]
            for cp in cps:
                cp.wait()

            def blend_body(g, c):
                pid = pbase + g * 16 + iota
                x = pid & (W - 1)
                y = pid >> 10
                posx = x.astype(jnp.float32) * scale
                posy = y.astype(jnp.float32) * scale
                x0f = posx.astype(jnp.int32).astype(jnp.float32)
                y0f = posy.astype(jnp.int32).astype(jnp.float32)
                wx = posx - x0f
                wy = posy - y0f
                ex = 1.0 - wx
                ey = 1.0 - wy
                sl = pl.ds(g * 16, 16)
                f0 = (r00a[sl] * ex + r10a[sl] * wx) * ey + \
                     (r01a[sl] * ex + r11a[sl] * wx) * wy
                f1 = (r00b[sl] * ex + r10b[sl] * wx) * ey + \
                     (r01b[sl] * ex + r11b[sl] * wx) * wy
                encv[pl.ds(2 * l * CH + g * 16, 16)] = f0
                encv[pl.ds((2 * l + 1) * CH + g * 16, 16)] = f1
                return c

            lax.fori_loop(0, NG16, blend_body, 0)
        pltpu.sync_copy(encv, enc.at[pl.ds((wid * NCHUNK + ci) * 32 * CH, 32 * CH)])
        return carry

    lax.fori_loop(0, NCHUNK, chunk_body, 0)


def _sc_encode(tbl):
    mesh = plsc.VectorSubcoreMesh(core_axis_name="c", subcore_axis_name="s")
    fn = pl.kernel(
        _sc_body,
        out_type=jax.ShapeDtypeStruct((TOTCH * 32 * CH,), jnp.float32),
        mesh=mesh,
        compiler_params=pltpu.CompilerParams(needs_layout_passes=False, use_tc_tiling_on_sc=False),
        scratch_types=[
            pltpu.VMEM((CH,), jnp.int32),
            pltpu.VMEM((CH,), jnp.int32),
            pltpu.VMEM((CH,), jnp.int32),
            pltpu.VMEM((CH,), jnp.int32),
            pltpu.VMEM((CH,), jnp.int32),
            pltpu.VMEM((CH,), jnp.int32),
            pltpu.VMEM((CH,), jnp.int32),
            pltpu.VMEM((CH,), jnp.int32),
            pltpu.VMEM((CH,), jnp.float32),
            pltpu.VMEM((CH,), jnp.float32),
            pltpu.VMEM((CH,), jnp.float32),
            pltpu.VMEM((CH,), jnp.float32),
            pltpu.VMEM((CH,), jnp.float32),
            pltpu.VMEM((CH,), jnp.float32),
            pltpu.VMEM((CH,), jnp.float32),
            pltpu.VMEM((CH,), jnp.float32),
            pltpu.VMEM((32 * CH,), jnp.float32),
            pltpu.SemaphoreType.DMA,
        ],
    )
    return fn(tbl)


ROWS_PER = CH // W


CPB = 8
ROWS_TC = CPB * ROWS_PER


def _tc_body(enc_ref, tex_ref, w1_ref, b1_ref, w2_ref, b2_ref,
             w3_ref, b3_ref, out_ref):
    base = 1.0 / (1.0 + jnp.exp(-tex_ref[0]))
    for s in range(CPB):
        e = enc_ref[pl.ds(s * 32 * CH, 32 * CH)].reshape(32, CH)
        e = e.astype(jnp.bfloat16)
        h1 = jnp.dot(w1_ref[...], e, preferred_element_type=jnp.float32)
        h1 = jnp.maximum(h1 + b1_ref[...], 0.0).astype(jnp.bfloat16)
        h2 = jnp.dot(w2_ref[...], h1, preferred_element_type=jnp.float32)
        h2 = jnp.maximum(h2 + b2_ref[...], 0.0).astype(jnp.bfloat16)
        r = jnp.dot(w3_ref[...], h2, preferred_element_type=jnp.float32)
        r = r + b3_ref[...]
        resid = RESIDUAL_SCALE * jnp.tanh(r)
        for rr in range(ROWS_PER):
            row = s * ROWS_PER + rr
            o = base[:, row, :] + resid[0:3, rr * W:(rr + 1) * W]
            out_ref[0, :, row, :] = jnp.clip(o, 0.0, 1.0)


def _tc_mlp(enc, texture_map, w1t, b1c, w2t, b2c, w3t, b3c):
    return pl.pallas_call(
        _tc_body,
        grid=(TOTCH // CPB,),
        in_specs=[
            pl.BlockSpec((CPB * 32 * CH,), lambda i: (i,)),
            pl.BlockSpec((1, 3, ROWS_TC, W), lambda i: (0, 0, i, 0)),
            pl.BlockSpec((64, 32), lambda i: (0, 0)),
            pl.BlockSpec((64, 1), lambda i: (0, 0)),
            pl.BlockSpec((64, 64), lambda i: (0, 0)),
            pl.BlockSpec((64, 1), lambda i: (0, 0)),
            pl.BlockSpec((8, 64), lambda i: (0, 0)),
            pl.BlockSpec((8, 1), lambda i: (0, 0)),
        ],
        out_specs=pl.BlockSpec((1, 3, ROWS_TC, W), lambda i: (0, 0, i, 0)),
        out_shape=jax.ShapeDtypeStruct((1, 3, H, W), jnp.float32),
    )(enc, texture_map, w1t, b1c, w2t, b2c, w3t, b3c)


def kernel(texture_map, hash_tables, W1, b1, W2, b2, W3, b3):
    tbl = hash_tables.reshape(L, T // 128, 128, F).transpose(0, 1, 3, 2).reshape(L * T * F)
    enc = _sc_encode(tbl)
    w1t = W1.T.astype(jnp.bfloat16)
    w2t = W2.T.astype(jnp.bfloat16)
    w3t = jnp.concatenate([W3.T, jnp.zeros((5, 64), W3.dtype)], axis=0)
    w3t = w3t.astype(jnp.bfloat16)
    b1c = b1.reshape(64, 1)
    b2c = b2.reshape(64, 1)
    b3c = jnp.concatenate([b3, jnp.zeros((5,), b3.dtype)]).reshape(8, 1)
    return _tc_mlp(enc, texture_map, w1t, b1c, w2t, b2c, w3t, b3c)


# speedup vs baseline: 212.6342x; 2.5152x over previous
import numpy as np
import jax
import jax.numpy as jnp
from jax import lax
from jax.experimental import pallas as pl
from jax.experimental.pallas import tpu as pltpu
from jax.experimental.pallas import tpu_sc as plsc

H = 1024
W = 1024
L = 16
T = 2 ** 19
F = 2
BASE_RES = 16
FINEST_RES = 2048
_b = np.exp((np.log(FINEST_RES) - np.log(BASE_RES)) / (L - 1))
RESOLUTIONS = [int(np.floor(BASE_RES * _b ** l)) for l in range(L)]
RESIDUAL_SCALE = 0.25
P1S = int(np.uint32(2654435761).astype(np.int64)) - (1 << 32)
P2S = 805459861
HMASK = T - 1

NC = 2
NS = 16
NWORK = NC * NS
PTS = H * W
PER_W = PTS // NWORK
CH = 1024
NCHUNK = PER_W // CH
TOTCH = PTS // CH
NG16 = CH // 16


def _sc_body(tbl, enc,
             i00a, i00b, i10a, i10b, i01a, i01b, i11a, i11b,
             r00a, r00b, r10a, r10b, r01a, r01b, r11a, r11b,
             encv, sem):
    wid = lax.axis_index("s") * NC + lax.axis_index("c")
    iota = lax.iota(jnp.int32, 16)

    def chunk_body(ci, carry):
        pbase = wid * PER_W + ci * CH
        yv = (pbase + iota) >> 10
        for l in range(L):
            res = RESOLUTIONS[l]
            scale = np.float32(res / 1024.0)
            ebase = 2 * l * T
            posy = yv.astype(jnp.float32) * scale
            y0 = posy.astype(jnp.int32)
            hy0 = y0 * P2S
            hy1 = hy0 + P2S
            wy = posy - y0.astype(jnp.float32)
            ey = 1.0 - wy

            if res < 1024:
                NJ = ((res + 1 + 15) // 16) * 16

                def didx_body(g, c):
                    j = g * 16 + iota
                    hx = j * P1S
                    h0 = (hx ^ hy0) & HMASK
                    h1 = (hx ^ hy1) & HMASK
                    e0 = ((h0 << 1) - (h0 & 127)) + ebase
                    e1 = ((h1 << 1) - (h1 & 127)) + ebase
                    sl = pl.ds(g * 16, 16)
                    i00a[sl] = e0
                    i00b[sl] = e0 + 128
                    i01a[sl] = e1
                    i01b[sl] = e1 + 128
                    return c

                lax.fori_loop(0, NJ // 16, didx_body, 0)
                cps = [
                    pltpu.async_copy(tbl.at[i00a.at[pl.ds(0, NJ)]],
                                     r00a.at[pl.ds(0, NJ)], sem),
                    pltpu.async_copy(tbl.at[i00b.at[pl.ds(0, NJ)]],
                                     r00b.at[pl.ds(0, NJ)], sem),
                    pltpu.async_copy(tbl.at[i01a.at[pl.ds(0, NJ)]],
                                     r01a.at[pl.ds(0, NJ)], sem),
                    pltpu.async_copy(tbl.at[i01b.at[pl.ds(0, NJ)]],
                                     r01b.at[pl.ds(0, NJ)], sem),
                ]
                for cp in cps:
                    cp.wait()

                def dblend_body(g, c):
                    x = g * 16 + iota
                    posx = x.astype(jnp.float32) * scale
                    x0 = posx.astype(jnp.int32)
                    wx = posx - x0.astype(jnp.float32)
                    ex = 1.0 - wx
                    x1 = x0 + 1
                    f00a = plsc.load_gather(r00a, [x0])
                    f10a = plsc.load_gather(r00a, [x1])
                    f01a = plsc.load_gather(r01a, [x0])
                    f11a = plsc.load_gather(r01a, [x1])
                    f00b = plsc.load_gather(r00b, [x0])
                    f10b = plsc.load_gather(r00b, [x1])
                    f01b = plsc.load_gather(r01b, [x0])
                    f11b = plsc.load_gather(r01b, [x1])
                    f0 = (f00a * ex + f10a * wx) * ey + \
                         (f01a * ex + f11a * wx) * wy
                    f1 = (f00b * ex + f10b * wx) * ey + \
                         (f01b * ex + f11b * wx) * wy
                    encv[pl.ds(2 * l * CH + g * 16, 16)] = f0
                    encv[pl.ds((2 * l + 1) * CH + g * 16, 16)] = f1
                    return c

                lax.fori_loop(0, NG16, dblend_body, 0)
            else:
                def idx_body(g, c):
                    x = g * 16 + iota
                    x0 = (x.astype(jnp.float32) * scale).astype(jnp.int32)
                    hx0 = x0 * P1S
                    hx1 = hx0 + P1S
                    sl = pl.ds(g * 16, 16)
                    h00 = (hx0 ^ hy0) & HMASK
                    h10 = (hx1 ^ hy0) & HMASK
                    h01 = (hx0 ^ hy1) & HMASK
                    h11 = (hx1 ^ hy1) & HMASK
                    e00 = ((h00 << 1) - (h00 & 127)) + ebase
                    e10 = ((h10 << 1) - (h10 & 127)) + ebase
                    e01 = ((h01 << 1) - (h01 & 127)) + ebase
                    e11 = ((h11 << 1) - (h11 & 127)) + ebase
                    i00a[sl] = e00
                    i00b[sl] = e00 + 128
                    i10a[sl] = e10
                    i10b[sl] = e10 + 128
                    i01a[sl] = e01
                    i01b[sl] = e01 + 128
                    i11a[sl] = e11
                    i11b[sl] = e11 + 128
                    return c

                lax.fori_loop(0, NG16, idx_body, 0)
                cps = [
                    pltpu.async_copy(tbl.at[i00a], r00a, sem),
                    pltpu.async_copy(tbl.at[i00b], r00b, sem),
                    pltpu.async_copy(tbl.at[i10a], r10a, sem),
                    pltpu.async_copy(tbl.at[i10b], r10b, sem),
                    pltpu.async_copy(tbl.at[i01a], r01a, sem),
                    pltpu.async_copy(tbl.at[i01b], r01b, sem),
                    pltpu.async_copy(tbl.at[i11a], r11a, sem),
                    pltpu.async_copy(tbl.at[i11b], r11b, sem),
                ]
                for cp in cps:
                    cp.wait()

                def blend_body(g, c):
                    x = g * 16 + iota
                    posx = x.astype(jnp.float32) * scale
                    x0f = posx.astype(jnp.int32).astype(jnp.float32)
                    wx = posx - x0f
                    ex = 1.0 - wx
                    sl = pl.ds(g * 16, 16)
                    f0 = (r00a[sl] * ex + r10a[sl] * wx) * ey + \
                         (r01a[sl] * ex + r11a[sl] * wx) * wy
                    f1 = (r00b[sl] * ex + r10b[sl] * wx) * ey + \
                         (r01b[sl] * ex + r11b[sl] * wx) * wy
                    encv[pl.ds(2 * l * CH + g * 16, 16)] = f0
                    encv[pl.ds((2 * l + 1) * CH + g * 16, 16)] = f1
                    return c

                lax.fori_loop(0, NG16, blend_body, 0)
        pltpu.sync_copy(encv, enc.at[pl.ds((wid * NCHUNK + ci) * 32 * CH, 32 * CH)])
        return carry

    lax.fori_loop(0, NCHUNK, chunk_body, 0)


def _sc_encode(tbl):
    mesh = plsc.VectorSubcoreMesh(core_axis_name="c", subcore_axis_name="s")
    fn = pl.kernel(
        _sc_body,
        out_type=jax.ShapeDtypeStruct((TOTCH * 32 * CH,), jnp.float32),
        mesh=mesh,
        compiler_params=pltpu.CompilerParams(needs_layout_passes=False, use_tc_tiling_on_sc=False),
        scratch_types=[
            pltpu.VMEM((CH,), jnp.int32),
            pltpu.VMEM((CH,), jnp.int32),
            pltpu.VMEM((CH,), jnp.int32),
            pltpu.VMEM((CH,), jnp.int32),
            pltpu.VMEM((CH,), jnp.int32),
            pltpu.VMEM((CH,), jnp.int32),
            pltpu.VMEM((CH,), jnp.int32),
            pltpu.VMEM((CH,), jnp.int32),
            pltpu.VMEM((CH,), jnp.float32),
            pltpu.VMEM((CH,), jnp.float32),
            pltpu.VMEM((CH,), jnp.float32),
            pltpu.VMEM((CH,), jnp.float32),
            pltpu.VMEM((CH,), jnp.float32),
            pltpu.VMEM((CH,), jnp.float32),
            pltpu.VMEM((CH,), jnp.float32),
            pltpu.VMEM((CH,), jnp.float32),
            pltpu.VMEM((32 * CH,), jnp.float32),
            pltpu.SemaphoreType.DMA,
        ],
    )
    return fn(tbl)


ROWS_PER = CH // W


CPB = 8
ROWS_TC = CPB * ROWS_PER


def _tc_body(enc_ref, tex_ref, w1_ref, b1_ref, w2_ref, b2_ref,
             w3_ref, b3_ref, out_ref):
    base = 1.0 / (1.0 + jnp.exp(-tex_ref[0]))
    for s in range(CPB):
        e = enc_ref[pl.ds(s * 32 * CH, 32 * CH)].reshape(32, CH)
        e = e.astype(jnp.bfloat16)
        h1 = jnp.dot(w1_ref[...], e, preferred_element_type=jnp.float32)
        h1 = jnp.maximum(h1 + b1_ref[...], 0.0).astype(jnp.bfloat16)
        h2 = jnp.dot(w2_ref[...], h1, preferred_element_type=jnp.float32)
        h2 = jnp.maximum(h2 + b2_ref[...], 0.0).astype(jnp.bfloat16)
        r = jnp.dot(w3_ref[...], h2, preferred_element_type=jnp.float32)
        r = r + b3_ref[...]
        resid = RESIDUAL_SCALE * jnp.tanh(r)
        for rr in range(ROWS_PER):
            row = s * ROWS_PER + rr
            o = base[:, row, :] + resid[0:3, rr * W:(rr + 1) * W]
            out_ref[0, :, row, :] = jnp.clip(o, 0.0, 1.0)


def _tc_mlp(enc, texture_map, w1t, b1c, w2t, b2c, w3t, b3c):
    return pl.pallas_call(
        _tc_body,
        grid=(TOTCH // CPB,),
        in_specs=[
            pl.BlockSpec((CPB * 32 * CH,), lambda i: (i,)),
            pl.BlockSpec((1, 3, ROWS_TC, W), lambda i: (0, 0, i, 0)),
            pl.BlockSpec((64, 32), lambda i: (0, 0)),
            pl.BlockSpec((64, 1), lambda i: (0, 0)),
            pl.BlockSpec((64, 64), lambda i: (0, 0)),
            pl.BlockSpec((64, 1), lambda i: (0, 0)),
            pl.BlockSpec((8, 64), lambda i: (0, 0)),
            pl.BlockSpec((8, 1), lambda i: (0, 0)),
        ],
        out_specs=pl.BlockSpec((1, 3, ROWS_TC, W), lambda i: (0, 0, i, 0)),
        out_shape=jax.ShapeDtypeStruct((1, 3, H, W), jnp.float32),
    )(enc, texture_map, w1t, b1c, w2t, b2c, w3t, b3c)


def kernel(texture_map, hash_tables, W1, b1, W2, b2, W3, b3):
    tbl = hash_tables.reshape(L, T // 128, 128, F).transpose(0, 1, 3, 2).reshape(L * T * F)
    enc = _sc_encode(tbl)
    w1t = W1.T.astype(jnp.bfloat16)
    w2t = W2.T.astype(jnp.bfloat16)
    w3t = jnp.concatenate([W3.T, jnp.zeros((5, 64), W3.dtype)], axis=0)
    w3t = w3t.astype(jnp.bfloat16)
    b1c = b1.reshape(64, 1)
    b2c = b2.reshape(64, 1)
    b3c = jnp.concatenate([b3, jnp.zeros((5,), b3.dtype)]).reshape(8, 1)
    return _tc_mlp(enc, texture_map, w1t, b1c, w2t, b2c, w3t, b3c)


# speedup vs baseline: 294.0611x; 1.3829x over previous
import numpy as np
import jax
import jax.numpy as jnp
from jax import lax
from jax.experimental import pallas as pl
from jax.experimental.pallas import tpu as pltpu
from jax.experimental.pallas import tpu_sc as plsc

H = 1024
W = 1024
L = 16
T = 2 ** 19
F = 2
BASE_RES = 16
FINEST_RES = 2048
_b = np.exp((np.log(FINEST_RES) - np.log(BASE_RES)) / (L - 1))
RESOLUTIONS = [int(np.floor(BASE_RES * _b ** l)) for l in range(L)]
RESIDUAL_SCALE = 0.25
P1S = int(np.uint32(2654435761).astype(np.int64)) - (1 << 32)
P2S = 805459861
HMASK = T - 1

NC = 2
NS = 16
NWORK = NC * NS
PTS = H * W
PER_W = PTS // NWORK
CH = 1024
NCHUNK = PER_W // CH
TOTCH = PTS // CH
NG16 = CH // 16


def _sc_body(tbl, enc, *scr):
    idx0, idx1 = list(scr[0:8]), list(scr[8:16])
    row0, row1 = list(scr[16:24]), list(scr[24:32])
    encv = scr[32]
    sem0, sem1 = scr[33], scr[34]
    wid = lax.axis_index("s") * NC + lax.axis_index("c")
    iota = lax.iota(jnp.int32, 16)

    def level_ctx(l, yv):
        res = RESOLUTIONS[l]
        scale = np.float32(res / 1024.0)
        posy = yv.astype(jnp.float32) * scale
        y0 = posy.astype(jnp.int32)
        hy0 = y0 * P2S
        wy = posy - y0.astype(jnp.float32)
        return scale, hy0, wy

    def fire(l, yv, idxs, rows, sem):
        res = RESOLUTIONS[l]
        scale, hy0, _ = level_ctx(l, yv)
        hy1 = hy0 + P2S
        ebase = 2 * l * T
        if res < 1024:
            NJ = ((res + 1 + 15) // 16) * 16
            ia0, ib0, ia1, ib1 = idxs[0], idxs[1], idxs[2], idxs[3]
            ra0, rb0, ra1, rb1 = rows[0], rows[1], rows[2], rows[3]

            def didx_body(g, c):
                j = g * 16 + iota
                hx = j * P1S
                h0 = (hx ^ hy0) & HMASK
                h1 = (hx ^ hy1) & HMASK
                e0 = ((h0 << 1) - (h0 & 127)) + ebase
                e1 = ((h1 << 1) - (h1 & 127)) + ebase
                sl = pl.ds(g * 16, 16)
                ia0[sl] = e0
                ib0[sl] = e0 + 128
                ia1[sl] = e1
                ib1[sl] = e1 + 128
                return c

            lax.fori_loop(0, NJ // 16, didx_body, 0)
            return [
                pltpu.async_copy(tbl.at[ia0.at[pl.ds(0, NJ)]],
                                 ra0.at[pl.ds(0, NJ)], sem),
                pltpu.async_copy(tbl.at[ib0.at[pl.ds(0, NJ)]],
                                 rb0.at[pl.ds(0, NJ)], sem),
                pltpu.async_copy(tbl.at[ia1.at[pl.ds(0, NJ)]],
                                 ra1.at[pl.ds(0, NJ)], sem),
                pltpu.async_copy(tbl.at[ib1.at[pl.ds(0, NJ)]],
                                 rb1.at[pl.ds(0, NJ)], sem),
            ]

        def idx_body(g, c):
            x = g * 16 + iota
            x0 = (x.astype(jnp.float32) * scale).astype(jnp.int32)
            hx0 = x0 * P1S
            hx1 = hx0 + P1S
            sl = pl.ds(g * 16, 16)
            h00 = (hx0 ^ hy0) & HMASK
            h10 = (hx1 ^ hy0) & HMASK
            h01 = (hx0 ^ hy1) & HMASK
            h11 = (hx1 ^ hy1) & HMASK
            e00 = ((h00 << 1) - (h00 & 127)) + ebase
            e10 = ((h10 << 1) - (h10 & 127)) + ebase
            e01 = ((h01 << 1) - (h01 & 127)) + ebase
            e11 = ((h11 << 1) - (h11 & 127)) + ebase
            idxs[0][sl] = e00
            idxs[1][sl] = e00 + 128
            idxs[2][sl] = e10
            idxs[3][sl] = e10 + 128
            idxs[4][sl] = e01
            idxs[5][sl] = e01 + 128
            idxs[6][sl] = e11
            idxs[7][sl] = e11 + 128
            return c

        lax.fori_loop(0, NG16, idx_body, 0)
        return [pltpu.async_copy(tbl.at[idxs[k]], rows[k], sem)
                for k in range(8)]

    def blend(l, yv, rows):
        res = RESOLUTIONS[l]
        scale, _, wy = level_ctx(l, yv)
        ey = 1.0 - wy
        if res < 1024:
            ra0, rb0, ra1, rb1 = rows[0], rows[1], rows[2], rows[3]

            def dblend_body(g, c):
                x = g * 16 + iota
                posx = x.astype(jnp.float32) * scale
                x0 = posx.astype(jnp.int32)
                wx = posx - x0.astype(jnp.float32)
                ex = 1.0 - wx
                x1 = x0 + 1
                f00a = plsc.load_gather(ra0, [x0])
                f10a = plsc.load_gather(ra0, [x1])
                f01a = plsc.load_gather(ra1, [x0])
                f11a = plsc.load_gather(ra1, [x1])
                f00b = plsc.load_gather(rb0, [x0])
                f10b = plsc.load_gather(rb0, [x1])
                f01b = plsc.load_gather(rb1, [x0])
                f11b = plsc.load_gather(rb1, [x1])
                f0 = (f00a * ex + f10a * wx) * ey + \
                     (f01a * ex + f11a * wx) * wy
                f1 = (f00b * ex + f10b * wx) * ey + \
                     (f01b * ex + f11b * wx) * wy
                encv[pl.ds(2 * l * CH + g * 16, 16)] = f0
                encv[pl.ds((2 * l + 1) * CH + g * 16, 16)] = f1
                return c

            lax.fori_loop(0, NG16, dblend_body, 0)
            return

        def blend_body(g, c):
            x = g * 16 + iota
            posx = x.astype(jnp.float32) * scale
            x0f = posx.astype(jnp.int32).astype(jnp.float32)
            wx = posx - x0f
            ex = 1.0 - wx
            sl = pl.ds(g * 16, 16)
            f0 = (rows[0][sl] * ex + rows[2][sl] * wx) * ey + \
                 (rows[4][sl] * ex + rows[6][sl] * wx) * wy
            f1 = (rows[1][sl] * ex + rows[3][sl] * wx) * ey + \
                 (rows[5][sl] * ex + rows[7][sl] * wx) * wy
            encv[pl.ds(2 * l * CH + g * 16, 16)] = f0
            encv[pl.ds((2 * l + 1) * CH + g * 16, 16)] = f1
            return c

        lax.fori_loop(0, NG16, blend_body, 0)

    def chunk_body(ci, carry):
        pbase = wid * PER_W + ci * CH
        yv = (pbase + iota) >> 10
        sets = [(idx0, row0, sem0), (idx1, row1, sem1)]
        prev = None
        for l in range(L):
            idxs, rows, sem = sets[l % 2]
            cps = fire(l, yv, idxs, rows, sem)
            if prev is not None:
                for cp in prev[0]:
                    cp.wait()
                blend(prev[1], yv, prev[2])
            prev = (cps, l, rows)
        for cp in prev[0]:
            cp.wait()
        blend(prev[1], yv, prev[2])
        pltpu.sync_copy(encv, enc.at[pl.ds((wid * NCHUNK + ci) * 32 * CH, 32 * CH)])
        return carry

    lax.fori_loop(0, NCHUNK, chunk_body, 0)


def _sc_encode(tbl):
    mesh = plsc.VectorSubcoreMesh(core_axis_name="c", subcore_axis_name="s")
    fn = pl.kernel(
        _sc_body,
        out_type=jax.ShapeDtypeStruct((TOTCH * 32 * CH,), jnp.float32),
        mesh=mesh,
        compiler_params=pltpu.CompilerParams(needs_layout_passes=False, use_tc_tiling_on_sc=False),
        scratch_types=(
            [pltpu.VMEM((CH,), jnp.int32)] * 16
            + [pltpu.VMEM((CH,), jnp.float32)] * 16
            + [pltpu.VMEM((32 * CH,), jnp.float32),
               pltpu.SemaphoreType.DMA,
               pltpu.SemaphoreType.DMA]
        ),
    )
    return fn(tbl)


ROWS_PER = CH // W


CPB = 8
ROWS_TC = CPB * ROWS_PER


def _tc_body(enc_ref, tex_ref, w1_ref, b1_ref, w2_ref, b2_ref,
             w3_ref, b3_ref, out_ref):
    base = 1.0 / (1.0 + jnp.exp(-tex_ref[0]))
    for s in range(CPB):
        e = enc_ref[pl.ds(s * 32 * CH, 32 * CH)].reshape(32, CH)
        e = e.astype(jnp.bfloat16)
        h1 = jnp.dot(w1_ref[...], e, preferred_element_type=jnp.float32)
        h1 = jnp.maximum(h1 + b1_ref[...], 0.0).astype(jnp.bfloat16)
        h2 = jnp.dot(w2_ref[...], h1, preferred_element_type=jnp.float32)
        h2 = jnp.maximum(h2 + b2_ref[...], 0.0).astype(jnp.bfloat16)
        r = jnp.dot(w3_ref[...], h2, preferred_element_type=jnp.float32)
        r = r + b3_ref[...]
        resid = RESIDUAL_SCALE * jnp.tanh(r)
        for rr in range(ROWS_PER):
            row = s * ROWS_PER + rr
            o = base[:, row, :] + resid[0:3, rr * W:(rr + 1) * W]
            out_ref[0, :, row, :] = jnp.clip(o, 0.0, 1.0)


def _tc_mlp(enc, texture_map, w1t, b1c, w2t, b2c, w3t, b3c):
    return pl.pallas_call(
        _tc_body,
        grid=(TOTCH // CPB,),
        in_specs=[
            pl.BlockSpec((CPB * 32 * CH,), lambda i: (i,)),
            pl.BlockSpec((1, 3, ROWS_TC, W), lambda i: (0, 0, i, 0)),
            pl.BlockSpec((64, 32), lambda i: (0, 0)),
            pl.BlockSpec((64, 1), lambda i: (0, 0)),
            pl.BlockSpec((64, 64), lambda i: (0, 0)),
            pl.BlockSpec((64, 1), lambda i: (0, 0)),
            pl.BlockSpec((8, 64), lambda i: (0, 0)),
            pl.BlockSpec((8, 1), lambda i: (0, 0)),
        ],
        out_specs=pl.BlockSpec((1, 3, ROWS_TC, W), lambda i: (0, 0, i, 0)),
        out_shape=jax.ShapeDtypeStruct((1, 3, H, W), jnp.float32),
    )(enc, texture_map, w1t, b1c, w2t, b2c, w3t, b3c)


def kernel(texture_map, hash_tables, W1, b1, W2, b2, W3, b3):
    tbl = hash_tables.reshape(L, T // 128, 128, F).transpose(0, 1, 3, 2).reshape(L * T * F)
    enc = _sc_encode(tbl)
    w1t = W1.T.astype(jnp.bfloat16)
    w2t = W2.T.astype(jnp.bfloat16)
    w3t = jnp.concatenate([W3.T, jnp.zeros((5, 64), W3.dtype)], axis=0)
    w3t = w3t.astype(jnp.bfloat16)
    b1c = b1.reshape(64, 1)
    b2c = b2.reshape(64, 1)
    b3c = jnp.concatenate([b3, jnp.zeros((5,), b3.dtype)]).reshape(8, 1)
    return _tc_mlp(enc, texture_map, w1t, b1c, w2t, b2c, w3t, b3c)


# speedup vs baseline: 301.4500x; 1.0251x over previous
import numpy as np
import jax
import jax.numpy as jnp
from jax import lax
from jax.experimental import pallas as pl
from jax.experimental.pallas import tpu as pltpu
from jax.experimental.pallas import tpu_sc as plsc

H = 1024
W = 1024
L = 16
T = 2 ** 19
F = 2
BASE_RES = 16
FINEST_RES = 2048
_b = np.exp((np.log(FINEST_RES) - np.log(BASE_RES)) / (L - 1))
RESOLUTIONS = [int(np.floor(BASE_RES * _b ** l)) for l in range(L)]
RESIDUAL_SCALE = 0.25
P1S = int(np.uint32(2654435761).astype(np.int64)) - (1 << 32)
P2S = 805459861
HMASK = T - 1

NC = 2
NS = 16
NWORK = NC * NS
PTS = H * W
PER_W = PTS // NWORK
CH = 1024
NCHUNK = PER_W // CH
TOTCH = PTS // CH
NG16 = CH // 16


def _sc_body(tbl, enc, *scr):
    idx0, idx1, idx2 = list(scr[0:8]), list(scr[8:16]), list(scr[16:24])
    row0, row1, row2 = list(scr[24:32]), list(scr[32:40]), list(scr[40:48])
    encv = scr[48]
    sem0, sem1, sem2 = scr[49], scr[50], scr[51]
    wid = lax.axis_index("s") * NC + lax.axis_index("c")
    iota = lax.iota(jnp.int32, 16)

    def level_ctx(l, yv):
        res = RESOLUTIONS[l]
        scale = np.float32(res / 1024.0)
        posy = yv.astype(jnp.float32) * scale
        y0 = posy.astype(jnp.int32)
        hy0 = y0 * P2S
        wy = posy - y0.astype(jnp.float32)
        return scale, hy0, wy

    def fire(l, yv, idxs, rows, sem):
        res = RESOLUTIONS[l]
        scale, hy0, _ = level_ctx(l, yv)
        hy1 = hy0 + P2S
        ebase = 2 * l * T
        if res < 1024:
            NJ = ((res + 1 + 15) // 16) * 16
            ia0, ib0, ia1, ib1 = idxs[0], idxs[1], idxs[2], idxs[3]
            ra0, rb0, ra1, rb1 = rows[0], rows[1], rows[2], rows[3]

            def didx_body(g, c):
                j = g * 16 + iota
                hx = j * P1S
                h0 = (hx ^ hy0) & HMASK
                h1 = (hx ^ hy1) & HMASK
                e0 = ((h0 << 1) - (h0 & 127)) + ebase
                e1 = ((h1 << 1) - (h1 & 127)) + ebase
                sl = pl.ds(g * 16, 16)
                ia0[sl] = e0
                ib0[sl] = e0 + 128
                ia1[sl] = e1
                ib1[sl] = e1 + 128
                return c

            lax.fori_loop(0, NJ // 16, didx_body, 0)
            return [
                pltpu.async_copy(tbl.at[ia0.at[pl.ds(0, NJ)]],
                                 ra0.at[pl.ds(0, NJ)], sem),
                pltpu.async_copy(tbl.at[ib0.at[pl.ds(0, NJ)]],
                                 rb0.at[pl.ds(0, NJ)], sem),
                pltpu.async_copy(tbl.at[ia1.at[pl.ds(0, NJ)]],
                                 ra1.at[pl.ds(0, NJ)], sem),
                pltpu.async_copy(tbl.at[ib1.at[pl.ds(0, NJ)]],
                                 rb1.at[pl.ds(0, NJ)], sem),
            ]

        def idx_body(g, c):
            x = g * 16 + iota
            x0 = (x.astype(jnp.float32) * scale).astype(jnp.int32)
            hx0 = x0 * P1S
            hx1 = hx0 + P1S
            sl = pl.ds(g * 16, 16)
            h00 = (hx0 ^ hy0) & HMASK
            h10 = (hx1 ^ hy0) & HMASK
            h01 = (hx0 ^ hy1) & HMASK
            h11 = (hx1 ^ hy1) & HMASK
            e00 = ((h00 << 1) - (h00 & 127)) + ebase
            e10 = ((h10 << 1) - (h10 & 127)) + ebase
            e01 = ((h01 << 1) - (h01 & 127)) + ebase
            e11 = ((h11 << 1) - (h11 & 127)) + ebase
            idxs[0][sl] = e00
            idxs[1][sl] = e00 + 128
            idxs[2][sl] = e10
            idxs[3][sl] = e10 + 128
            idxs[4][sl] = e01
            idxs[5][sl] = e01 + 128
            idxs[6][sl] = e11
            idxs[7][sl] = e11 + 128
            return c

        lax.fori_loop(0, NG16, idx_body, 0)
        return [pltpu.async_copy(tbl.at[idxs[k]], rows[k], sem)
                for k in range(8)]

    def blend(l, yv, rows):
        res = RESOLUTIONS[l]
        scale, _, wy = level_ctx(l, yv)
        ey = 1.0 - wy
        if res < 1024:
            ra0, rb0, ra1, rb1 = rows[0], rows[1], rows[2], rows[3]

            def dblend_body(g, c):
                x = g * 16 + iota
                posx = x.astype(jnp.float32) * scale
                x0 = posx.astype(jnp.int32)
                wx = posx - x0.astype(jnp.float32)
                ex = 1.0 - wx
                x1 = x0 + 1
                f00a = plsc.load_gather(ra0, [x0])
                f10a = plsc.load_gather(ra0, [x1])
                f01a = plsc.load_gather(ra1, [x0])
                f11a = plsc.load_gather(ra1, [x1])
                f00b = plsc.load_gather(rb0, [x0])
                f10b = plsc.load_gather(rb0, [x1])
                f01b = plsc.load_gather(rb1, [x0])
                f11b = plsc.load_gather(rb1, [x1])
                f0 = (f00a * ex + f10a * wx) * ey + \
                     (f01a * ex + f11a * wx) * wy
                f1 = (f00b * ex + f10b * wx) * ey + \
                     (f01b * ex + f11b * wx) * wy
                encv[pl.ds(2 * l * CH + g * 16, 16)] = f0
                encv[pl.ds((2 * l + 1) * CH + g * 16, 16)] = f1
                return c

            lax.fori_loop(0, NG16, dblend_body, 0)
            return

        def blend_body(g, c):
            x = g * 16 + iota
            posx = x.astype(jnp.float32) * scale
            x0f = posx.astype(jnp.int32).astype(jnp.float32)
            wx = posx - x0f
            ex = 1.0 - wx
            sl = pl.ds(g * 16, 16)
            f0 = (rows[0][sl] * ex + rows[2][sl] * wx) * ey + \
                 (rows[4][sl] * ex + rows[6][sl] * wx) * wy
            f1 = (rows[1][sl] * ex + rows[3][sl] * wx) * ey + \
                 (rows[5][sl] * ex + rows[7][sl] * wx) * wy
            encv[pl.ds(2 * l * CH + g * 16, 16)] = f0
            encv[pl.ds((2 * l + 1) * CH + g * 16, 16)] = f1
            return c

        lax.fori_loop(0, NG16, blend_body, 0)

    def chunk_body(ci, carry):
        pbase = wid * PER_W + ci * CH
        yv = (pbase + iota) >> 10
        sets = [(idx0, row0, sem0), (idx1, row1, sem1), (idx2, row2, sem2)]
        pending = []
        for l in range(L):
            idxs, rows, sem = sets[l % 3]
            cps = fire(l, yv, idxs, rows, sem)
            pending.append((cps, l, rows))
            if len(pending) > 2:
                cps0, l0, rows0 = pending.pop(0)
                for cp in cps0:
                    cp.wait()
                blend(l0, yv, rows0)
        for cps0, l0, rows0 in pending:
            for cp in cps0:
                cp.wait()
            blend(l0, yv, rows0)
        pltpu.sync_copy(encv, enc.at[pl.ds((wid * NCHUNK + ci) * 32 * CH, 32 * CH)])
        return carry

    lax.fori_loop(0, NCHUNK, chunk_body, 0)


def _sc_encode(tbl):
    mesh = plsc.VectorSubcoreMesh(core_axis_name="c", subcore_axis_name="s")
    fn = pl.kernel(
        _sc_body,
        out_type=jax.ShapeDtypeStruct((TOTCH * 32 * CH,), jnp.float32),
        mesh=mesh,
        compiler_params=pltpu.CompilerParams(needs_layout_passes=False, use_tc_tiling_on_sc=False),
        scratch_types=(
            [pltpu.VMEM((CH,), jnp.int32)] * 24
            + [pltpu.VMEM((CH,), jnp.float32)] * 24
            + [pltpu.VMEM((32 * CH,), jnp.float32),
               pltpu.SemaphoreType.DMA,
               pltpu.SemaphoreType.DMA,
               pltpu.SemaphoreType.DMA]
        ),
    )
    return fn(tbl)


ROWS_PER = CH // W


CPB = 8
ROWS_TC = CPB * ROWS_PER


def _tc_body(enc_ref, tex_ref, w1_ref, b1_ref, w2_ref, b2_ref,
             w3_ref, b3_ref, out_ref):
    base = 1.0 / (1.0 + jnp.exp(-tex_ref[0]))
    for s in range(CPB):
        e = enc_ref[pl.ds(s * 32 * CH, 32 * CH)].reshape(32, CH)
        e = e.astype(jnp.bfloat16)
        h1 = jnp.dot(w1_ref[...], e, preferred_element_type=jnp.float32)
        h1 = jnp.maximum(h1 + b1_ref[...], 0.0).astype(jnp.bfloat16)
        h2 = jnp.dot(w2_ref[...], h1, preferred_element_type=jnp.float32)
        h2 = jnp.maximum(h2 + b2_ref[...], 0.0).astype(jnp.bfloat16)
        r = jnp.dot(w3_ref[...], h2, preferred_element_type=jnp.float32)
        r = r + b3_ref[...]
        resid = RESIDUAL_SCALE * jnp.tanh(r)
        for rr in range(ROWS_PER):
            row = s * ROWS_PER + rr
            o = base[:, row, :] + resid[0:3, rr * W:(rr + 1) * W]
            out_ref[0, :, row, :] = jnp.clip(o, 0.0, 1.0)


def _tc_mlp(enc, texture_map, w1t, b1c, w2t, b2c, w3t, b3c):
    return pl.pallas_call(
        _tc_body,
        grid=(TOTCH // CPB,),
        in_specs=[
            pl.BlockSpec((CPB * 32 * CH,), lambda i: (i,)),
            pl.BlockSpec((1, 3, ROWS_TC, W), lambda i: (0, 0, i, 0)),
            pl.BlockSpec((64, 32), lambda i: (0, 0)),
            pl.BlockSpec((64, 1), lambda i: (0, 0)),
            pl.BlockSpec((64, 64), lambda i: (0, 0)),
            pl.BlockSpec((64, 1), lambda i: (0, 0)),
            pl.BlockSpec((8, 64), lambda i: (0, 0)),
            pl.BlockSpec((8, 1), lambda i: (0, 0)),
        ],
        out_specs=pl.BlockSpec((1, 3, ROWS_TC, W), lambda i: (0, 0, i, 0)),
        out_shape=jax.ShapeDtypeStruct((1, 3, H, W), jnp.float32),
    )(enc, texture_map, w1t, b1c, w2t, b2c, w3t, b3c)


def kernel(texture_map, hash_tables, W1, b1, W2, b2, W3, b3):
    tbl = hash_tables.reshape(L, T // 128, 128, F).transpose(0, 1, 3, 2).reshape(L * T * F)
    enc = _sc_encode(tbl)
    w1t = W1.T.astype(jnp.bfloat16)
    w2t = W2.T.astype(jnp.bfloat16)
    w3t = jnp.concatenate([W3.T, jnp.zeros((5, 64), W3.dtype)], axis=0)
    w3t = w3t.astype(jnp.bfloat16)
    b1c = b1.reshape(64, 1)
    b2c = b2.reshape(64, 1)
    b3c = jnp.concatenate([b3, jnp.zeros((5,), b3.dtype)]).reshape(8, 1)
    return _tc_mlp(enc, texture_map, w1t, b1c, w2t, b2c, w3t, b3c)


# speedup vs baseline: 348.5129x; 1.1561x over previous
import numpy as np
import jax
import jax.numpy as jnp
from jax import lax
from jax.experimental import pallas as pl
from jax.experimental.pallas import tpu as pltpu
from jax.experimental.pallas import tpu_sc as plsc

H = 1024
W = 1024
L = 16
T = 2 ** 19
F = 2
BASE_RES = 16
FINEST_RES = 2048
_b = np.exp((np.log(FINEST_RES) - np.log(BASE_RES)) / (L - 1))
RESOLUTIONS = [int(np.floor(BASE_RES * _b ** l)) for l in range(L)]
RESIDUAL_SCALE = 0.25
P1S = int(np.uint32(2654435761).astype(np.int64)) - (1 << 32)
P2S = 805459861
HMASK = T - 1

NC = 2
NS = 16
NWORK = NC * NS
PTS = H * W
PER_W = PTS // NWORK
CH = 1024
NCHUNK = PER_W // CH
TOTCH = PTS // CH
NG16 = CH // 16
NJBUF = 1488


def _sc_body(tbl, enc, *scr):
    idx0, idx1, idx2 = list(scr[0:8]), list(scr[8:16]), list(scr[16:24])
    row0, row1, row2 = list(scr[24:32]), list(scr[32:40]), list(scr[40:48])
    encv = scr[48]
    sem0, sem1, sem2 = scr[49], scr[50], scr[51]
    wid = lax.axis_index("s") * NC + lax.axis_index("c")
    iota = lax.iota(jnp.int32, 16)

    def level_ctx(l, yv):
        res = RESOLUTIONS[l]
        scale = np.float32(res / 1024.0)
        posy = yv.astype(jnp.float32) * scale
        y0 = posy.astype(jnp.int32)
        hy0 = y0 * P2S
        wy = posy - y0.astype(jnp.float32)
        return scale, hy0, wy

    def fire(l, yv, idxs, rows, sem):
        res = RESOLUTIONS[l]
        scale, hy0, _ = level_ctx(l, yv)
        hy1 = hy0 + P2S
        ebase = 2 * l * T
        if res + 1 <= NJBUF:
            NJ = ((res + 1 + 15) // 16) * 16
            ia0, ib0, ia1, ib1 = idxs[0], idxs[1], idxs[2], idxs[3]
            ra0, rb0, ra1, rb1 = rows[0], rows[1], rows[2], rows[3]

            def didx_body(g, c):
                j = g * 16 + iota
                hx = j * P1S
                h0 = (hx ^ hy0) & HMASK
                h1 = (hx ^ hy1) & HMASK
                e0 = ((h0 << 1) - (h0 & 127)) + ebase
                e1 = ((h1 << 1) - (h1 & 127)) + ebase
                sl = pl.ds(g * 16, 16)
                ia0[sl] = e0
                ib0[sl] = e0 + 128
                ia1[sl] = e1
                ib1[sl] = e1 + 128
                return c

            lax.fori_loop(0, NJ // 16, didx_body, 0)
            return [
                pltpu.async_copy(tbl.at[ia0.at[pl.ds(0, NJ)]],
                                 ra0.at[pl.ds(0, NJ)], sem),
                pltpu.async_copy(tbl.at[ib0.at[pl.ds(0, NJ)]],
                                 rb0.at[pl.ds(0, NJ)], sem),
                pltpu.async_copy(tbl.at[ia1.at[pl.ds(0, NJ)]],
                                 ra1.at[pl.ds(0, NJ)], sem),
                pltpu.async_copy(tbl.at[ib1.at[pl.ds(0, NJ)]],
                                 rb1.at[pl.ds(0, NJ)], sem),
            ]

        def idx_body(g, c):
            x = g * 16 + iota
            x0 = (x.astype(jnp.float32) * scale).astype(jnp.int32)
            hx0 = x0 * P1S
            hx1 = hx0 + P1S
            sl = pl.ds(g * 16, 16)
            h00 = (hx0 ^ hy0) & HMASK
            h10 = (hx1 ^ hy0) & HMASK
            h01 = (hx0 ^ hy1) & HMASK
            h11 = (hx1 ^ hy1) & HMASK
            e00 = ((h00 << 1) - (h00 & 127)) + ebase
            e10 = ((h10 << 1) - (h10 & 127)) + ebase
            e01 = ((h01 << 1) - (h01 & 127)) + ebase
            e11 = ((h11 << 1) - (h11 & 127)) + ebase
            idxs[0][sl] = e00
            idxs[1][sl] = e00 + 128
            idxs[2][sl] = e10
            idxs[3][sl] = e10 + 128
            idxs[4][sl] = e01
            idxs[5][sl] = e01 + 128
            idxs[6][sl] = e11
            idxs[7][sl] = e11 + 128
            return c

        lax.fori_loop(0, NG16, idx_body, 0)
        return [pltpu.async_copy(tbl.at[idxs[k].at[pl.ds(0, CH)]],
                                 rows[k].at[pl.ds(0, CH)], sem)
                for k in range(8)]

    def blend(l, yv, rows):
        res = RESOLUTIONS[l]
        scale, _, wy = level_ctx(l, yv)
        ey = 1.0 - wy
        if res + 1 <= NJBUF:
            ra0, rb0, ra1, rb1 = rows[0], rows[1], rows[2], rows[3]

            def dblend_body(g, c):
                x = g * 16 + iota
                posx = x.astype(jnp.float32) * scale
                x0 = posx.astype(jnp.int32)
                wx = posx - x0.astype(jnp.float32)
                ex = 1.0 - wx
                x1 = x0 + 1
                f00a = plsc.load_gather(ra0, [x0])
                f10a = plsc.load_gather(ra0, [x1])
                f01a = plsc.load_gather(ra1, [x0])
                f11a = plsc.load_gather(ra1, [x1])
                f00b = plsc.load_gather(rb0, [x0])
                f10b = plsc.load_gather(rb0, [x1])
                f01b = plsc.load_gather(rb1, [x0])
                f11b = plsc.load_gather(rb1, [x1])
                f0 = (f00a * ex + f10a * wx) * ey + \
                     (f01a * ex + f11a * wx) * wy
                f1 = (f00b * ex + f10b * wx) * ey + \
                     (f01b * ex + f11b * wx) * wy
                encv[pl.ds(2 * l * CH + g * 16, 16)] = f0
                encv[pl.ds((2 * l + 1) * CH + g * 16, 16)] = f1
                return c

            lax.fori_loop(0, NG16, dblend_body, 0)
            return

        def blend_body(g, c):
            x = g * 16 + iota
            posx = x.astype(jnp.float32) * scale
            x0f = posx.astype(jnp.int32).astype(jnp.float32)
            wx = posx - x0f
            ex = 1.0 - wx
            sl = pl.ds(g * 16, 16)
            f0 = (rows[0][sl] * ex + rows[2][sl] * wx) * ey + \
                 (rows[4][sl] * ex + rows[6][sl] * wx) * wy
            f1 = (rows[1][sl] * ex + rows[3][sl] * wx) * ey + \
                 (rows[5][sl] * ex + rows[7][sl] * wx) * wy
            encv[pl.ds(2 * l * CH + g * 16, 16)] = f0
            encv[pl.ds((2 * l + 1) * CH + g * 16, 16)] = f1
            return c

        lax.fori_loop(0, NG16, blend_body, 0)

    def chunk_body(ci, carry):
        pbase = wid * PER_W + ci * CH
        yv = (pbase + iota) >> 10
        sets = [(idx0, row0, sem0), (idx1, row1, sem1), (idx2, row2, sem2)]
        pending = []
        for l in range(L):
            idxs, rows, sem = sets[l % 3]
            cps = fire(l, yv, idxs, rows, sem)
            pending.append((cps, l, rows))
            if len(pending) > 2:
                cps0, l0, rows0 = pending.pop(0)
                for cp in cps0:
                    cp.wait()
                blend(l0, yv, rows0)
        for cps0, l0, rows0 in pending:
            for cp in cps0:
                cp.wait()
            blend(l0, yv, rows0)
        pltpu.sync_copy(encv, enc.at[pl.ds((wid * NCHUNK + ci) * 32 * CH, 32 * CH)])
        return carry

    lax.fori_loop(0, NCHUNK, chunk_body, 0)


def _sc_encode(tbl):
    mesh = plsc.VectorSubcoreMesh(core_axis_name="c", subcore_axis_name="s")
    fn = pl.kernel(
        _sc_body,
        out_type=jax.ShapeDtypeStruct((TOTCH * 32 * CH,), jnp.float32),
        mesh=mesh,
        compiler_params=pltpu.CompilerParams(needs_layout_passes=False, use_tc_tiling_on_sc=False),
        scratch_types=(
            [pltpu.VMEM((NJBUF,), jnp.int32)] * 24
            + [pltpu.VMEM((NJBUF,), jnp.float32)] * 24
            + [pltpu.VMEM((32 * CH,), jnp.float32),
               pltpu.SemaphoreType.DMA,
               pltpu.SemaphoreType.DMA,
               pltpu.SemaphoreType.DMA]
        ),
    )
    return fn(tbl)


ROWS_PER = CH // W


CPB = 8
ROWS_TC = CPB * ROWS_PER


def _tc_body(enc_ref, tex_ref, w1_ref, b1_ref, w2_ref, b2_ref,
             w3_ref, b3_ref, out_ref):
    base = 1.0 / (1.0 + jnp.exp(-tex_ref[0]))
    for s in range(CPB):
        e = enc_ref[pl.ds(s * 32 * CH, 32 * CH)].reshape(32, CH)
        e = e.astype(jnp.bfloat16)
        h1 = jnp.dot(w1_ref[...], e, preferred_element_type=jnp.float32)
        h1 = jnp.maximum(h1 + b1_ref[...], 0.0).astype(jnp.bfloat16)
        h2 = jnp.dot(w2_ref[...], h1, preferred_element_type=jnp.float32)
        h2 = jnp.maximum(h2 + b2_ref[...], 0.0).astype(jnp.bfloat16)
        r = jnp.dot(w3_ref[...], h2, preferred_element_type=jnp.float32)
        r = r + b3_ref[...]
        resid = RESIDUAL_SCALE * jnp.tanh(r)
        for rr in range(ROWS_PER):
            row = s * ROWS_PER + rr
            o = base[:, row, :] + resid[0:3, rr * W:(rr + 1) * W]
            out_ref[0, :, row, :] = jnp.clip(o, 0.0, 1.0)


def _tc_mlp(enc, texture_map, w1t, b1c, w2t, b2c, w3t, b3c):
    return pl.pallas_call(
        _tc_body,
        grid=(TOTCH // CPB,),
        in_specs=[
            pl.BlockSpec((CPB * 32 * CH,), lambda i: (i,)),
            pl.BlockSpec((1, 3, ROWS_TC, W), lambda i: (0, 0, i, 0)),
            pl.BlockSpec((64, 32), lambda i: (0, 0)),
            pl.BlockSpec((64, 1), lambda i: (0, 0)),
            pl.BlockSpec((64, 64), lambda i: (0, 0)),
            pl.BlockSpec((64, 1), lambda i: (0, 0)),
            pl.BlockSpec((8, 64), lambda i: (0, 0)),
            pl.BlockSpec((8, 1), lambda i: (0, 0)),
        ],
        out_specs=pl.BlockSpec((1, 3, ROWS_TC, W), lambda i: (0, 0, i, 0)),
        out_shape=jax.ShapeDtypeStruct((1, 3, H, W), jnp.float32),
    )(enc, texture_map, w1t, b1c, w2t, b2c, w3t, b3c)


def kernel(texture_map, hash_tables, W1, b1, W2, b2, W3, b3):
    tbl = hash_tables.reshape(L, T // 128, 128, F).transpose(0, 1, 3, 2).reshape(L * T * F)
    enc = _sc_encode(tbl)
    w1t = W1.T.astype(jnp.bfloat16)
    w2t = W2.T.astype(jnp.bfloat16)
    w3t = jnp.concatenate([W3.T, jnp.zeros((5, 64), W3.dtype)], axis=0)
    w3t = w3t.astype(jnp.bfloat16)
    b1c = b1.reshape(64, 1)
    b2c = b2.reshape(64, 1)
    b3c = jnp.concatenate([b3, jnp.zeros((5,), b3.dtype)]).reshape(8, 1)
    return _tc_mlp(enc, texture_map, w1t, b1c, w2t, b2c, w3t, b3c)
